# baseline (speedup 1.0000x reference)
import functools

import jax
import jax.numpy as jnp
from jax import lax
from jax.experimental import pallas as pl
from jax.experimental.pallas import tpu as pltpu
from jax.experimental.pallas import tpu_sc as plsc

F32 = jnp.float32
N, C, PP = 3136, 96, 16
NPAD = 3200
GPAD = 3584
TEMP = 0.001
G8 = 8
NROW2 = N * PP


def _unfold(img):
    return img.reshape(C, 56, 4, 56, 4).transpose(1, 3, 2, 4, 0).reshape(N, PP, C)


def _pool_body(t_ref, r_ref, x1_ref, x2_ref):
    for src, dst in ((t_ref, x1_ref), (r_ref, x2_ref)):
        x = src[...]
        acc = x[:, 0, :]
        for j in range(1, PP):
            acc = acc + x[:, j, :]
        dst[...] = acc * (1.0 / PP)


def _unfold_pool_body(t_ref, r_ref, a_ref, tf_ref, rf_ref, af_ref,
                      x1_ref, x2_ref):
    ys = []
    for src, dst in ((t_ref, tf_ref), (r_ref, rf_ref), (a_ref, af_ref)):
        z = src[...].reshape(C, 4 * 224)
        zt = jnp.swapaxes(z, 0, 1)
        y = zt.reshape(4, 56, 4, C).transpose(1, 0, 2, 3)
        dst[...] = y
        ys.append(y)
    for y, dst in ((ys[0], x1_ref), (ys[1], x2_ref)):
        acc = y[:, 0, 0, :]
        for u in range(4):
            for v in range(4):
                if u or v:
                    acc = acc + y[:, u, v, :]
        dst[...] = acc * (1.0 / PP)


def _dist_body(x1_ref, x2_ref, idx_ref):
    x1 = x1_ref[...]
    x2 = x2_ref[...]
    x1n = jnp.sum(x1 * x1, axis=1, keepdims=True)
    x2n = jnp.sum(x2 * x2, axis=1)
    g = lax.dot_general(x1.astype(jnp.bfloat16), x2.astype(jnp.bfloat16),
                        (((1,), (1,)), ((), ())),
                        preferred_element_type=F32)
    d2 = x1n + x2n[None, :] - 2.0 * g
    d = jnp.sqrt(jnp.clip(d2, 1e-30, None))
    rows = lax.broadcasted_iota(jnp.int32, d.shape, 0)
    m = jnp.min(d, axis=0, keepdims=True)
    cand = jnp.where(d <= m, rows, jnp.int32(2**30))
    idx_ref[0, 0, :] = jnp.clip(jnp.min(cand, axis=0), 0, N - 1)


def _combine_body(t_ref, r_ref, a_ref, o_ref):
    T = t_ref[...]
    Rr = r_ref[...]
    A = a_ref[...]
    tn = jnp.sum(T * T, axis=1, keepdims=True)
    rn = jnp.sum(Rr * Rr, axis=1)
    g = lax.dot_general(T, Rr, (((1,), (1,)), ((), ())),
                        preferred_element_type=F32,
                        precision=lax.Precision.HIGHEST)
    e = tn + rn[None, :] - 2.0 * g
    d = jnp.sqrt(jnp.clip(e, 1e-30, None))
    bx = lax.broadcasted_iota(jnp.int32, d.shape, 0) // PP
    by = lax.broadcasted_iota(jnp.int32, d.shape, 1) // PP
    dm = jnp.where(bx == by, d, 1e30)
    z = -dm / TEMP
    mz = jnp.max(z, axis=1, keepdims=True)
    ez = jnp.exp(z - mz)
    s = ez / jnp.sum(ez, axis=1, keepdims=True)
    o_ref[...] = lax.dot_general(s.astype(jnp.bfloat16), A.astype(jnp.bfloat16),
                                 (((1,), (0,)), ((), ())),
                                 preferred_element_type=F32)


def _fold_body(o_ref, img_ref):
    o = o_ref[...]
    y = o.transpose(1, 0, 2, 3).reshape(4 * 224, C)
    img_ref[...] = jnp.swapaxes(y, 0, 1).reshape(C, 1, 4, 224)


def _make_gather():
    info = plsc.get_sparse_core_info()
    nc = info.num_cores
    bpw = GPAD // (nc * info.num_subcores)
    ch = 16
    nch = bpw // ch
    mesh = plsc.VectorSubcoreMesh(core_axis_name="c", subcore_axis_name="s")

    @functools.partial(
        pl.kernel, mesh=mesh,
        out_type=[jax.ShapeDtypeStruct((GPAD, PP * C), F32)] * 2,
        scratch_types=[
            pltpu.VMEM((bpw,), jnp.int32),
            pltpu.VMEM((ch, PP * C), F32),
            pltpu.VMEM((ch, PP * C), F32),
            pltpu.SemaphoreType.DMA,
            pltpu.SemaphoreType.DMA,
            pltpu.SemaphoreType.DMA,
            pltpu.SemaphoreType.DMA,
        ],
    )
    def gather_k(rf_hbm, af_hbm, idx_hbm, outr_hbm, outa_hbm,
                 idx_v, rows_a, rows_b, gs_a, gs_b, ws_a, ws_b):
        wid = lax.axis_index("s") * nc + lax.axis_index("c")
        base = wid * bpw
        pltpu.sync_copy(idx_hbm.at[pl.ds(base, bpw)], idx_v)
        rows = (rows_a, rows_b)
        gsem = (gs_a, gs_b)
        wsem = (ws_a, ws_b)
        steps = [(tbl, out, k) for tbl, out in
                 ((rf_hbm, outr_hbm), (af_hbm, outa_hbm)) for k in range(nch)]
        pending = [None, None]
        for i, (tbl, out, k) in enumerate(steps):
            b = i % 2
            if pending[b] is not None:
                pending[b].wait()
            pltpu.async_copy(tbl.at[idx_v.at[pl.ds(k * ch, ch)]],
                             rows[b], gsem[b]).wait()
            pending[b] = pltpu.async_copy(rows[b], out.at[pl.ds(base + k * ch, ch)],
                                          wsem[b])
        for p in pending:
            p.wait()

    return gather_k


_gather = _make_gather()


def kernel(target, ref, ref_align):
    tgt4 = target.reshape(C, 56, 4, 224)
    ref4 = ref.reshape(C, 56, 4, 224)
    ra4 = ref_align.reshape(C, 56, 4, 224)

    img_spec = pl.BlockSpec((C, 1, 4, 224), lambda i: (0, i, 0, 0))
    tbl_spec = pl.BlockSpec((56, 4, 4, C), lambda i: (i, 0, 0, 0))
    Tf4, Rf4, Af4, x1, x2 = pl.pallas_call(
        _unfold_pool_body,
        grid=(56,),
        in_specs=[img_spec, img_spec, img_spec],
        out_specs=[tbl_spec] * 3 +
                  [pl.BlockSpec((56, C), lambda i: (i, 0))] * 2,
        out_shape=[jax.ShapeDtypeStruct((N, 4, 4, C), F32)] * 3 +
                  [jax.ShapeDtypeStruct((N, C), F32)] * 2,
    )(tgt4, ref4, ra4)
    Tf = Tf4.reshape(N, PP, C)
    Rf = Rf4.reshape(N, PP, C)
    Af = Af4.reshape(N, PP, C)

    idx3 = pl.pallas_call(
        _dist_body,
        grid=(GPAD // 64,),
        in_specs=[pl.BlockSpec((N, C), lambda i: (0, 0)),
                  pl.BlockSpec((64, C), lambda i: (i, 0))],
        out_specs=pl.BlockSpec((1, 1, 64), lambda i: (i, 0, 0)),
        out_shape=jax.ShapeDtypeStruct((GPAD // 64, 1, 64), jnp.int32),
    )(x1, x2)
    idxp = idx3.reshape(GPAD)

    Rg, Ag = _gather(Rf.reshape(N, PP * C), Af.reshape(N, PP * C), idxp)

    O2 = pl.pallas_call(
        _combine_body,
        grid=(N // G8,),
        in_specs=[pl.BlockSpec((G8 * PP, C), lambda i: (i, 0))] * 3,
        out_specs=pl.BlockSpec((G8 * PP, C), lambda i: (i, 0)),
        out_shape=jax.ShapeDtypeStruct((NROW2, C), F32),
    )(Tf.reshape(NROW2, C),
      Rg.reshape(GPAD * PP, C),
      Ag.reshape(GPAD * PP, C))

    out4 = pl.pallas_call(
        _fold_body,
        grid=(56,),
        in_specs=[pl.BlockSpec((56, 4, 4, C), lambda i: (i, 0, 0, 0))],
        out_specs=pl.BlockSpec((C, 1, 4, 224), lambda i: (0, i, 0, 0)),
        out_shape=jax.ShapeDtypeStruct((C, 56, 4, 224), F32),
    )(O2.reshape(N, 4, 4, C))
    return out4.reshape(1, C, 224, 224)


# speedup vs baseline: 1.0350x; 1.0350x over previous
import functools

import jax
import jax.numpy as jnp
from jax import lax
from jax.experimental import pallas as pl
from jax.experimental.pallas import tpu as pltpu
from jax.experimental.pallas import tpu_sc as plsc

F32 = jnp.float32
N, C, PP = 3136, 96, 16
NPAD = 3200
GPAD = 3584
TEMP = 0.001
G8 = 8
NROW2 = N * PP


def _unfold(img):
    return img.reshape(C, 56, 4, 56, 4).transpose(1, 3, 2, 4, 0).reshape(N, PP, C)


def _pool_body(t_ref, r_ref, x1_ref, x2_ref):
    for src, dst in ((t_ref, x1_ref), (r_ref, x2_ref)):
        x = src[...]
        acc = x[:, 0, :]
        for j in range(1, PP):
            acc = acc + x[:, j, :]
        dst[...] = acc * (1.0 / PP)


def _unfold_pool_body(t_ref, r_ref, a_ref, tf_ref, rf_ref, af_ref,
                      x1_ref, x2_ref):
    ys = []
    for src, dst in ((t_ref, tf_ref), (r_ref, rf_ref), (a_ref, af_ref)):
        z = src[...].reshape(C, 4 * 224)
        zt = jnp.swapaxes(z, 0, 1)
        y = zt.reshape(4, 56, 4, C).transpose(1, 0, 2, 3)
        dst[...] = y
        ys.append(y)
    for y, dst in ((ys[0], x1_ref), (ys[1], x2_ref)):
        acc = y[:, 0, 0, :]
        for u in range(4):
            for v in range(4):
                if u or v:
                    acc = acc + y[:, u, v, :]
        dst[...] = acc * (1.0 / PP)


def _dist_body(x1_ref, x2_ref, idx_ref):
    x1 = x1_ref[...]
    x2 = x2_ref[...]
    x1n = jnp.sum(x1 * x1, axis=1, keepdims=True)
    x2n = jnp.sum(x2 * x2, axis=1)
    g = lax.dot_general(x1.astype(jnp.bfloat16), x2.astype(jnp.bfloat16),
                        (((1,), (1,)), ((), ())),
                        preferred_element_type=F32)
    d2 = x1n + x2n[None, :] - 2.0 * g
    d = jnp.sqrt(jnp.clip(d2, 1e-30, None))
    rows = lax.broadcasted_iota(jnp.int32, d.shape, 0)
    m = jnp.min(d, axis=0, keepdims=True)
    cand = jnp.where(d <= m, rows, jnp.int32(2**30))
    idx_ref[0, 0, :] = jnp.clip(jnp.min(cand, axis=0), 0, N - 1)


def _combine_body(t_ref, r_ref, a_ref, o_ref):
    T = t_ref[...]
    Rr = r_ref[...]
    A = a_ref[...]
    tn = jnp.sum(T * T, axis=1, keepdims=True)
    rn = jnp.sum(Rr * Rr, axis=1)
    g = lax.dot_general(T, Rr, (((1,), (1,)), ((), ())),
                        preferred_element_type=F32,
                        precision=lax.Precision.HIGHEST)
    e = tn + rn[None, :] - 2.0 * g
    d = jnp.sqrt(jnp.clip(e, 1e-30, None))
    bx = lax.broadcasted_iota(jnp.int32, d.shape, 0) // PP
    by = lax.broadcasted_iota(jnp.int32, d.shape, 1) // PP
    dm = jnp.where(bx == by, d, 1e30)
    z = -dm / TEMP
    mz = jnp.max(z, axis=1, keepdims=True)
    ez = jnp.exp(z - mz)
    s = ez / jnp.sum(ez, axis=1, keepdims=True)
    o_ref[...] = lax.dot_general(s.astype(jnp.bfloat16), A.astype(jnp.bfloat16),
                                 (((1,), (0,)), ((), ())),
                                 preferred_element_type=F32)


def _fold_body(o_ref, img_ref):
    o = o_ref[...]
    y = o.transpose(1, 0, 2, 3).reshape(4 * 224, C)
    img_ref[...] = jnp.swapaxes(y, 0, 1).reshape(C, 1, 4, 224)


def _make_gather():
    info = plsc.get_sparse_core_info()
    nc = info.num_cores
    bpw = GPAD // (nc * info.num_subcores)
    ch = 16
    nch = bpw // ch
    mesh = plsc.VectorSubcoreMesh(core_axis_name="c", subcore_axis_name="s")

    @functools.partial(
        pl.kernel, mesh=mesh,
        compiler_params=pltpu.CompilerParams(use_tc_tiling_on_sc=True),
        out_type=[jax.ShapeDtypeStruct((GPAD, PP * C), F32)] * 2,
        scratch_types=[
            pltpu.VMEM((bpw,), jnp.int32),
            pltpu.VMEM((ch, PP * C), F32),
            pltpu.VMEM((ch, PP * C), F32),
            pltpu.SemaphoreType.DMA,
            pltpu.SemaphoreType.DMA,
            pltpu.SemaphoreType.DMA,
            pltpu.SemaphoreType.DMA,
        ],
    )
    def gather_k(rf_hbm, af_hbm, idx_hbm, outr_hbm, outa_hbm,
                 idx_v, rows_a, rows_b, gs_a, gs_b, ws_a, ws_b):
        wid = lax.axis_index("s") * nc + lax.axis_index("c")
        base = wid * bpw
        pltpu.sync_copy(idx_hbm.at[pl.ds(base, bpw)], idx_v)
        rows = (rows_a, rows_b)
        gsem = (gs_a, gs_b)
        wsem = (ws_a, ws_b)
        steps = [(tbl, out, k) for tbl, out in
                 ((rf_hbm, outr_hbm), (af_hbm, outa_hbm)) for k in range(nch)]
        pending = [None, None]
        for i, (tbl, out, k) in enumerate(steps):
            b = i % 2
            if pending[b] is not None:
                pending[b].wait()
            pltpu.async_copy(tbl.at[idx_v.at[pl.ds(k * ch, ch)]],
                             rows[b], gsem[b]).wait()
            pending[b] = pltpu.async_copy(rows[b], out.at[pl.ds(base + k * ch, ch)],
                                          wsem[b])
        for p in pending:
            p.wait()

    return gather_k


_gather = _make_gather()


def kernel(target, ref, ref_align):
    tgt4 = target.reshape(C, 56, 4, 224)
    ref4 = ref.reshape(C, 56, 4, 224)
    ra4 = ref_align.reshape(C, 56, 4, 224)

    img_spec = pl.BlockSpec((C, 1, 4, 224), lambda i: (0, i, 0, 0))
    tbl_spec = pl.BlockSpec((56, 4, 4, C), lambda i: (i, 0, 0, 0))
    Tf4, Rf4, Af4, x1, x2 = pl.pallas_call(
        _unfold_pool_body,
        grid=(56,),
        in_specs=[img_spec, img_spec, img_spec],
        out_specs=[tbl_spec] * 3 +
                  [pl.BlockSpec((56, C), lambda i: (i, 0))] * 2,
        out_shape=[jax.ShapeDtypeStruct((N, 4, 4, C), F32)] * 3 +
                  [jax.ShapeDtypeStruct((N, C), F32)] * 2,
    )(tgt4, ref4, ra4)
    Tf = Tf4.reshape(N, PP, C)
    Rf = Rf4.reshape(N, PP, C)
    Af = Af4.reshape(N, PP, C)

    idx3 = pl.pallas_call(
        _dist_body,
        grid=(GPAD // 64,),
        in_specs=[pl.BlockSpec((N, C), lambda i: (0, 0)),
                  pl.BlockSpec((64, C), lambda i: (i, 0))],
        out_specs=pl.BlockSpec((1, 1, 64), lambda i: (i, 0, 0)),
        out_shape=jax.ShapeDtypeStruct((GPAD // 64, 1, 64), jnp.int32),
    )(x1, x2)
    idxp = idx3.reshape(GPAD)

    Rg, Ag = _gather(Rf.reshape(N, PP * C), Af.reshape(N, PP * C), idxp)

    O2 = pl.pallas_call(
        _combine_body,
        grid=(N // G8,),
        in_specs=[pl.BlockSpec((G8 * PP, C), lambda i: (i, 0))] * 3,
        out_specs=pl.BlockSpec((G8 * PP, C), lambda i: (i, 0)),
        out_shape=jax.ShapeDtypeStruct((NROW2, C), F32),
    )(Tf.reshape(NROW2, C),
      Rg.reshape(GPAD * PP, C),
      Ag.reshape(GPAD * PP, C))

    out = O2.reshape(56, 56, 4, 4, C).transpose(4, 0, 2, 1, 3).reshape(1, C, 224, 224)
    return out


# speedup vs baseline: 1.0503x; 1.0149x over previous
import functools

import jax
import jax.numpy as jnp
from jax import lax
from jax.experimental import pallas as pl
from jax.experimental.pallas import tpu as pltpu
from jax.experimental.pallas import tpu_sc as plsc

F32 = jnp.float32
N, C, PP = 3136, 96, 16
NPAD = 3200
GPAD = 3584
TEMP = 0.001
G8 = 8
NROW2 = N * PP


def _unfold(img):
    return img.reshape(C, 56, 4, 56, 4).transpose(1, 3, 2, 4, 0).reshape(N, PP, C)


def _pool_body(t_ref, r_ref, x1_ref, x2_ref):
    for src, dst in ((t_ref, x1_ref), (r_ref, x2_ref)):
        x = src[...]
        acc = x[:, 0, :]
        for j in range(1, PP):
            acc = acc + x[:, j, :]
        dst[...] = acc * (1.0 / PP)


def _unfold_pool_body(t_ref, r_ref, a_ref, tf_ref, rf_ref, af_ref,
                      x1_ref, x2_ref):
    ys = []
    for src, dst in ((t_ref, tf_ref), (r_ref, rf_ref), (a_ref, af_ref)):
        z = src[...].reshape(C, 4 * 224)
        zt = jnp.swapaxes(z, 0, 1)
        y = zt.reshape(4, 56, 4, C).transpose(1, 0, 2, 3)
        dst[...] = y
        ys.append(y)
    for y, dst in ((ys[0], x1_ref), (ys[1], x2_ref)):
        acc = y[:, 0, 0, :]
        for u in range(4):
            for v in range(4):
                if u or v:
                    acc = acc + y[:, u, v, :]
        dst[...] = acc * (1.0 / PP)


def _dist_body(x1_ref, x2_ref, idx_ref):
    x1 = x1_ref[...]
    x2 = x2_ref[...]
    x1n = jnp.sum(x1 * x1, axis=1, keepdims=True)
    x2n = jnp.sum(x2 * x2, axis=1)
    g = lax.dot_general(x1.astype(jnp.bfloat16), x2.astype(jnp.bfloat16),
                        (((1,), (1,)), ((), ())),
                        preferred_element_type=F32)
    d2 = x1n + x2n[None, :] - 2.0 * g
    d = jnp.sqrt(jnp.clip(d2, 1e-30, None))
    rows = lax.broadcasted_iota(jnp.int32, d.shape, 0)
    m = jnp.min(d, axis=0, keepdims=True)
    cand = jnp.where(d <= m, rows, jnp.int32(2**30))
    idx_ref[0, 0, :] = jnp.clip(jnp.min(cand, axis=0), 0, N - 1)


def _combine_body(t_ref, r_ref, a_ref, o_ref):
    T = t_ref[...]
    Rr = r_ref[...]
    A = a_ref[...]
    tn = jnp.sum(T * T, axis=1, keepdims=True)
    rn = jnp.sum(Rr * Rr, axis=1)
    nt = (((1,), (1,)), ((), ()))
    thi = T.astype(jnp.bfloat16)
    tlo = (T - thi.astype(F32)).astype(jnp.bfloat16)
    rhi = Rr.astype(jnp.bfloat16)
    rlo = (Rr - rhi.astype(F32)).astype(jnp.bfloat16)
    g = (lax.dot_general(thi, rhi, nt, preferred_element_type=F32)
         + lax.dot_general(thi, rlo, nt, preferred_element_type=F32)
         + lax.dot_general(tlo, rhi, nt, preferred_element_type=F32))
    e = tn + rn[None, :] - 2.0 * g
    d = jnp.sqrt(jnp.clip(e, 1e-30, None))
    bx = lax.broadcasted_iota(jnp.int32, d.shape, 0) // PP
    by = lax.broadcasted_iota(jnp.int32, d.shape, 1) // PP
    dm = jnp.where(bx == by, d, 1e30)
    z = -dm / TEMP
    mz = jnp.max(z, axis=1, keepdims=True)
    ez = jnp.exp(z - mz)
    s = ez / jnp.sum(ez, axis=1, keepdims=True)
    o_ref[...] = lax.dot_general(s.astype(jnp.bfloat16), A.astype(jnp.bfloat16),
                                 (((1,), (0,)), ((), ())),
                                 preferred_element_type=F32)


def _fold_body(o_ref, img_ref):
    o = o_ref[...]
    y = o.transpose(1, 0, 2, 3).reshape(4 * 224, C)
    img_ref[...] = jnp.swapaxes(y, 0, 1).reshape(C, 1, 4, 224)


def _make_gather():
    info = plsc.get_sparse_core_info()
    nc = info.num_cores
    bpw = GPAD // (nc * info.num_subcores)
    ch = 16
    nch = bpw // ch
    mesh = plsc.VectorSubcoreMesh(core_axis_name="c", subcore_axis_name="s")

    @functools.partial(
        pl.kernel, mesh=mesh,
        out_type=[jax.ShapeDtypeStruct((GPAD, PP * C), F32)] * 2,
        scratch_types=[
            pltpu.VMEM((bpw,), jnp.int32),
            pltpu.VMEM((ch, PP * C), F32),
            pltpu.VMEM((ch, PP * C), F32),
            pltpu.SemaphoreType.DMA,
            pltpu.SemaphoreType.DMA,
            pltpu.SemaphoreType.DMA,
            pltpu.SemaphoreType.DMA,
        ],
    )
    def gather_k(rf_hbm, af_hbm, idx_hbm, outr_hbm, outa_hbm,
                 idx_v, rows_a, rows_b, gs_a, gs_b, ws_a, ws_b):
        wid = lax.axis_index("s") * nc + lax.axis_index("c")
        base = wid * bpw
        pltpu.sync_copy(idx_hbm.at[pl.ds(base, bpw)], idx_v)
        rows = (rows_a, rows_b)
        gsem = (gs_a, gs_b)
        wsem = (ws_a, ws_b)
        steps = [(tbl, out, k) for tbl, out in
                 ((rf_hbm, outr_hbm), (af_hbm, outa_hbm)) for k in range(nch)]
        pending = [None, None]
        for i, (tbl, out, k) in enumerate(steps):
            b = i % 2
            if pending[b] is not None:
                pending[b].wait()
            pltpu.async_copy(tbl.at[idx_v.at[pl.ds(k * ch, ch)]],
                             rows[b], gsem[b]).wait()
            pending[b] = pltpu.async_copy(rows[b], out.at[pl.ds(base + k * ch, ch)],
                                          wsem[b])
        for p in pending:
            p.wait()

    return gather_k


_gather = _make_gather()


def kernel(target, ref, ref_align):
    tgt4 = target.reshape(C, 56, 4, 224)
    ref4 = ref.reshape(C, 56, 4, 224)
    ra4 = ref_align.reshape(C, 56, 4, 224)

    img_spec = pl.BlockSpec((C, 1, 4, 224), lambda i: (0, i, 0, 0))
    tbl_spec = pl.BlockSpec((56, 4, 4, C), lambda i: (i, 0, 0, 0))
    Tf4, Rf4, Af4, x1, x2 = pl.pallas_call(
        _unfold_pool_body,
        grid=(56,),
        in_specs=[img_spec, img_spec, img_spec],
        out_specs=[tbl_spec] * 3 +
                  [pl.BlockSpec((56, C), lambda i: (i, 0))] * 2,
        out_shape=[jax.ShapeDtypeStruct((N, 4, 4, C), F32)] * 3 +
                  [jax.ShapeDtypeStruct((N, C), F32)] * 2,
    )(tgt4, ref4, ra4)
    Tf = Tf4.reshape(N, PP, C)
    Rf = Rf4.reshape(N, PP, C)
    Af = Af4.reshape(N, PP, C)

    idx3 = pl.pallas_call(
        _dist_body,
        grid=(GPAD // 64,),
        in_specs=[pl.BlockSpec((N, C), lambda i: (0, 0)),
                  pl.BlockSpec((64, C), lambda i: (i, 0))],
        out_specs=pl.BlockSpec((1, 1, 64), lambda i: (i, 0, 0)),
        out_shape=jax.ShapeDtypeStruct((GPAD // 64, 1, 64), jnp.int32),
    )(x1, x2)
    idxp = idx3.reshape(GPAD)

    Rg, Ag = _gather(Rf.reshape(N, PP * C), Af.reshape(N, PP * C), idxp)

    O2 = pl.pallas_call(
        _combine_body,
        grid=(N // G8,),
        in_specs=[pl.BlockSpec((G8 * PP, C), lambda i: (i, 0))] * 3,
        out_specs=pl.BlockSpec((G8 * PP, C), lambda i: (i, 0)),
        out_shape=jax.ShapeDtypeStruct((NROW2, C), F32),
    )(Tf.reshape(NROW2, C),
      Rg.reshape(GPAD * PP, C),
      Ag.reshape(GPAD * PP, C))

    out = O2.reshape(56, 56, 4, 4, C).transpose(4, 0, 2, 1, 3).reshape(1, C, 224, 224)
    return out


# speedup vs baseline: 1.0510x; 1.0006x over previous
import functools

import jax
import jax.numpy as jnp
from jax import lax
from jax.experimental import pallas as pl
from jax.experimental.pallas import tpu as pltpu
from jax.experimental.pallas import tpu_sc as plsc

F32 = jnp.float32
N, C, PP = 3136, 96, 16
GPAD = 3584
TEMP = 0.001
G8 = 8
NROW2 = N * PP


def _unfold_pool_body(t_ref, r_ref, a_ref, tf_ref, rf_ref, af_ref,
                      x1_ref, x2_ref):
    ys = []
    for src, dst in ((t_ref, tf_ref), (r_ref, rf_ref), (a_ref, af_ref)):
        z = src[...].reshape(C, 4 * 224)
        zt = jnp.swapaxes(z, 0, 1)
        y = zt.reshape(4, 56, 4, C).transpose(1, 0, 2, 3)
        dst[...] = y
        ys.append(y)
    for y, dst in ((ys[0], x1_ref), (ys[1], x2_ref)):
        acc = y[:, 0, 0, :]
        for u in range(4):
            for v in range(4):
                if u or v:
                    acc = acc + y[:, u, v, :]
        dst[...] = acc * (1.0 / PP)


def _dist_body(x1_ref, x2_ref, idx_ref):
    x1 = x1_ref[...]
    x2 = x2_ref[...]
    x1n = jnp.sum(x1 * x1, axis=1, keepdims=True)
    x2n = jnp.sum(x2 * x2, axis=1)
    g = lax.dot_general(x1.astype(jnp.bfloat16), x2.astype(jnp.bfloat16),
                        (((1,), (1,)), ((), ())),
                        preferred_element_type=F32)
    d2 = x1n + x2n[None, :] - 2.0 * g
    d = jnp.sqrt(jnp.clip(d2, 1e-30, None))
    rows = lax.broadcasted_iota(jnp.int32, d.shape, 0)
    m = jnp.min(d, axis=0, keepdims=True)
    cand = jnp.where(d <= m, rows, jnp.int32(2**30))
    idx_ref[0, 0, :] = jnp.clip(jnp.min(cand, axis=0), 0, N - 1)


def _combine_body(t_ref, r_ref, a_ref, o_ref):
    T = t_ref[...]
    Rr = r_ref[...]
    A = a_ref[...]
    tn = jnp.sum(T * T, axis=1, keepdims=True)
    rn = jnp.sum(Rr * Rr, axis=1)
    nt = (((1,), (1,)), ((), ()))
    thi = T.astype(jnp.bfloat16)
    tlo = (T - thi.astype(F32)).astype(jnp.bfloat16)
    rhi = Rr.astype(jnp.bfloat16)
    rlo = (Rr - rhi.astype(F32)).astype(jnp.bfloat16)
    g = (lax.dot_general(thi, rhi, nt, preferred_element_type=F32)
         + lax.dot_general(thi, rlo, nt, preferred_element_type=F32)
         + lax.dot_general(tlo, rhi, nt, preferred_element_type=F32))
    e = tn + rn[None, :] - 2.0 * g
    d = jnp.sqrt(jnp.clip(e, 1e-30, None))
    bx = lax.broadcasted_iota(jnp.int32, d.shape, 0) // PP
    by = lax.broadcasted_iota(jnp.int32, d.shape, 1) // PP
    dm = jnp.where(bx == by, d, 1e30)
    z = -dm / TEMP
    mz = jnp.max(z, axis=1, keepdims=True)
    ez = jnp.exp(z - mz)
    s = ez / jnp.sum(ez, axis=1, keepdims=True)
    o_ref[...] = lax.dot_general(s.astype(jnp.bfloat16), A.astype(jnp.bfloat16),
                                 (((1,), (0,)), ((), ())),
                                 preferred_element_type=F32)


def _make_gather():
    info = plsc.get_sparse_core_info()
    nc = info.num_cores
    bpw = GPAD // (nc * info.num_subcores)
    ch = 16
    nch = bpw // ch
    mesh = plsc.VectorSubcoreMesh(core_axis_name="c", subcore_axis_name="s")

    @functools.partial(
        pl.kernel, mesh=mesh,
        out_type=[jax.ShapeDtypeStruct((GPAD, PP * C), F32)] * 2,
        scratch_types=[
            pltpu.VMEM((bpw,), jnp.int32),
            pltpu.VMEM((ch, PP * C), F32),
            pltpu.VMEM((ch, PP * C), F32),
            pltpu.SemaphoreType.DMA,
            pltpu.SemaphoreType.DMA,
            pltpu.SemaphoreType.DMA,
            pltpu.SemaphoreType.DMA,
        ],
    )
    def gather_k(rf_hbm, af_hbm, idx_hbm, outr_hbm, outa_hbm,
                 idx_v, rows_a, rows_b, gs_a, gs_b, ws_a, ws_b):
        wid = lax.axis_index("s") * nc + lax.axis_index("c")
        base = wid * bpw
        pltpu.sync_copy(idx_hbm.at[pl.ds(base, bpw)], idx_v)
        rows = (rows_a, rows_b)
        gsem = (gs_a, gs_b)
        wsem = (ws_a, ws_b)
        steps = [(tbl, out, k) for tbl, out in
                 ((rf_hbm, outr_hbm), (af_hbm, outa_hbm)) for k in range(nch)]
        pending = [None, None]
        for i, (tbl, out, k) in enumerate(steps):
            b = i % 2
            if pending[b] is not None:
                pending[b].wait()
            pltpu.async_copy(tbl.at[idx_v.at[pl.ds(k * ch, ch)]],
                             rows[b], gsem[b]).wait()
            pending[b] = pltpu.async_copy(rows[b], out.at[pl.ds(base + k * ch, ch)],
                                          wsem[b])
        for p in pending:
            p.wait()

    return gather_k


_gather = _make_gather()


def kernel(target, ref, ref_align):
    tgt4 = target.reshape(C, 56, 4, 224)
    ref4 = ref.reshape(C, 56, 4, 224)
    ra4 = ref_align.reshape(C, 56, 4, 224)

    img_spec = pl.BlockSpec((C, 1, 4, 224), lambda i: (0, i, 0, 0))
    tbl_spec = pl.BlockSpec((56, 4, 4, C), lambda i: (i, 0, 0, 0))
    Tf4, Rf4, Af4, x1, x2 = pl.pallas_call(
        _unfold_pool_body,
        grid=(56,),
        in_specs=[img_spec, img_spec, img_spec],
        out_specs=[tbl_spec] * 3 +
                  [pl.BlockSpec((56, C), lambda i: (i, 0))] * 2,
        out_shape=[jax.ShapeDtypeStruct((N, 4, 4, C), F32)] * 3 +
                  [jax.ShapeDtypeStruct((N, C), F32)] * 2,
    )(tgt4, ref4, ra4)
    Tf = Tf4.reshape(N, PP, C)
    Rf = Rf4.reshape(N, PP, C)
    Af = Af4.reshape(N, PP, C)

    idx3 = pl.pallas_call(
        _dist_body,
        grid=(GPAD // 64,),
        in_specs=[pl.BlockSpec((N, C), lambda i: (0, 0)),
                  pl.BlockSpec((64, C), lambda i: (i, 0))],
        out_specs=pl.BlockSpec((1, 1, 64), lambda i: (i, 0, 0)),
        out_shape=jax.ShapeDtypeStruct((GPAD // 64, 1, 64), jnp.int32),
    )(x1, x2)
    idxp = idx3.reshape(GPAD)

    Rg, Ag = _gather(Rf.reshape(N, PP * C), Af.reshape(N, PP * C), idxp)

    O2 = pl.pallas_call(
        _combine_body,
        grid=(N // G8,),
        in_specs=[pl.BlockSpec((G8 * PP, C), lambda i: (i, 0))] * 3,
        out_specs=pl.BlockSpec((G8 * PP, C), lambda i: (i, 0)),
        out_shape=jax.ShapeDtypeStruct((NROW2, C), F32),
    )(Tf.reshape(NROW2, C),
      Rg.reshape(GPAD * PP, C),
      Ag.reshape(GPAD * PP, C))

    out = O2.reshape(56, 56, 4, 4, C).transpose(4, 0, 2, 1, 3).reshape(1, C, 224, 224)
    return out
